# baseline (speedup 1.0000x reference)
import functools

import jax
import jax.numpy as jnp
from jax import lax
from jax.experimental import pallas as pl
from jax.experimental.pallas import tpu as pltpu
from jax.experimental.pallas import tpu_sc as plsc

D = 128
LANES = 16
NC = 2
NS = 16
NW = NC * NS
BATCH = 1024
SEQ = 200
ROWS_PER_W = BATCH // NW
HALF = SEQ // 2
NBUF = 3

_mesh = plsc.VectorSubcoreMesh(core_axis_name="c", subcore_axis_name="s")


@functools.partial(
    pl.kernel,
    mesh=_mesh,
    out_type=jax.ShapeDtypeStruct((BATCH, SEQ, D), jnp.float32),
    scratch_types=[
        pltpu.VMEM((ROWS_PER_W, 2, HALF), jnp.int32),
        pltpu.VMEM((SEQ, D), jnp.float32),
        pltpu.VMEM((SEQ, D), jnp.float32),
        pltpu.VMEM((SEQ, D), jnp.float32),
        pltpu.VMEM((SEQ, D), jnp.float32),
        pltpu.SemaphoreType.DMA,
        pltpu.SemaphoreType.DMA,
        pltpu.SemaphoreType.DMA,
        pltpu.SemaphoreType.DMA,
        pltpu.SemaphoreType.DMA,
        pltpu.SemaphoreType.DMA,
        pltpu.SemaphoreType.DMA,
    ],
)
def _emb_lookup(x_hbm, wtab_hbm, pos_hbm, out_hbm,
                idx_v, pos_v, buf0, buf1, buf2, g0, g1, g2, s0, s1, s2, psem):
    wid = lax.axis_index("s") * NC + lax.axis_index("c")
    base_row = wid * ROWS_PER_W

    pltpu.sync_copy(x_hbm.at[wid], idx_v)
    pos_copy = pltpu.async_copy(pos_hbm, pos_v, psem)

    bufs = [buf0, buf1, buf2]
    gsems = [g0, g1, g2]
    ssems = [s0, s1, s2]
    pend_gather = [None] * NBUF
    pend_store = [None] * NBUF

    def start_gather(c, b):
        ga = pltpu.async_copy(
            wtab_hbm.at[idx_v.at[c, 0]],
            bufs[b].at[pl.ds(0, HALF)], gsems[b])
        gb = pltpu.async_copy(
            wtab_hbm.at[idx_v.at[c, 1]],
            bufs[b].at[pl.ds(HALF, HALF)], gsems[b])
        pend_gather[b] = (ga, gb)

    def add_pos(b, lo):
        buf = bufs[b]

        def body(r, carry):
            for u in range(2):
                for j in range(D // LANES):
                    sl = pl.ds(j * LANES, LANES)
                    plsc.addupdate(buf.at[lo + r * 2 + u, sl],
                                   pos_v[lo + r * 2 + u, sl])
            return carry

        lax.fori_loop(0, HALF // 2, body, 0)

    start_gather(0, 0)
    start_gather(1, 1)
    pos_copy.wait()

    for c in range(ROWS_PER_W):
        b = c % NBUF
        pend_gather[b][0].wait()
        pend_gather[b][1].wait()
        add_pos(b, 0)
        add_pos(b, HALF)
        pend_store[b] = pltpu.async_copy(
            bufs[b], out_hbm.at[base_row + c], ssems[b])
        if c + 2 < ROWS_PER_W:
            nb = (c + 2) % NBUF
            if pend_store[nb] is not None:
                pend_store[nb].wait()
                pend_store[nb] = None
            start_gather(c + 2, nb)

    for b in range(NBUF):
        if pend_store[b] is not None:
            pend_store[b].wait()


def _mask_body(x_ref, o_ref):
    o_ref[...] = x_ref[...] != 0


_mask_tc = pl.pallas_call(
    _mask_body,
    out_shape=jax.ShapeDtypeStruct((BATCH, SEQ), jnp.bool_),
)


@jax.jit
def kernel(x, word_emb, pos_emb):
    x_idx = x.reshape(NW, ROWS_PER_W, 2, HALF)
    out = _emb_lookup(x_idx, word_emb, pos_emb)
    mask = _mask_tc(x)
    return (out, mask)


# speedup vs baseline: 1.0388x; 1.0388x over previous
import functools

import jax
import jax.numpy as jnp
from jax import lax
from jax.experimental import pallas as pl
from jax.experimental.pallas import tpu as pltpu
from jax.experimental.pallas import tpu_sc as plsc

D = 128
LANES = 16
NC = 2
NS = 16
NW = NC * NS
BATCH = 1024
SEQ = 200
ROWS_PER_W = BATCH // NW
HALF = SEQ // 2
NBUF = 3

_mesh = plsc.VectorSubcoreMesh(core_axis_name="c", subcore_axis_name="s")


@functools.partial(
    pl.kernel,
    mesh=_mesh,
    out_type=jax.ShapeDtypeStruct((BATCH, SEQ, D), jnp.float32),
    scratch_types=[
        pltpu.VMEM((ROWS_PER_W, 2, HALF), jnp.int32),
        pltpu.VMEM((SEQ, D), jnp.float32),
        pltpu.VMEM((SEQ, D), jnp.float32),
        pltpu.VMEM((SEQ, D), jnp.float32),
        pltpu.VMEM((SEQ, D), jnp.float32),
        pltpu.SemaphoreType.DMA,
        pltpu.SemaphoreType.DMA,
        pltpu.SemaphoreType.DMA,
        pltpu.SemaphoreType.DMA,
        pltpu.SemaphoreType.DMA,
        pltpu.SemaphoreType.DMA,
        pltpu.SemaphoreType.DMA,
    ],
)
def _emb_lookup(x_hbm, wtab_hbm, pos_hbm, out_hbm,
                idx_v, pos_v, buf0, buf1, buf2, g0, g1, g2, s0, s1, s2, psem):
    wid = lax.axis_index("s") * NC + lax.axis_index("c")
    base_row = wid * ROWS_PER_W

    pltpu.sync_copy(x_hbm.at[wid], idx_v)
    pos_copy = pltpu.async_copy(pos_hbm, pos_v, psem)

    bufs = [buf0, buf1, buf2]
    gsems = [g0, g1, g2]
    ssems = [s0, s1, s2]
    pend_gather = [None] * NBUF
    pend_store = [None] * NBUF

    def start_gather(c, b):
        ga = pltpu.async_copy(
            wtab_hbm.at[idx_v.at[c, 0]],
            bufs[b].at[pl.ds(0, HALF)], gsems[b])
        gb = pltpu.async_copy(
            wtab_hbm.at[idx_v.at[c, 1]],
            bufs[b].at[pl.ds(HALF, HALF)], gsems[b])
        pend_gather[b] = (ga, gb)

    def add_pos(b):
        buf = bufs[b]

        def body(r, carry):
            for u in range(2):
                for j in range(D // LANES):
                    sl = pl.ds(j * LANES, LANES)
                    plsc.addupdate(buf.at[r * 2 + u, sl], pos_v[r * 2 + u, sl])
            return carry

        lax.fori_loop(0, SEQ // 2, body, 0)

    start_gather(0, 0)
    start_gather(1, 1)
    pos_copy.wait()

    for c in range(ROWS_PER_W):
        b = c % NBUF
        pend_gather[b][0].wait()
        pend_gather[b][1].wait()
        add_pos(b)
        pend_store[b] = pltpu.async_copy(
            bufs[b], out_hbm.at[base_row + c], ssems[b])
        if c + 2 < ROWS_PER_W:
            nb = (c + 2) % NBUF
            if pend_store[nb] is not None:
                pend_store[nb].wait()
                pend_store[nb] = None
            start_gather(c + 2, nb)

    for b in range(NBUF):
        if pend_store[b] is not None:
            pend_store[b].wait()


def _mask_body(x_ref, o_ref):
    o_ref[...] = x_ref[...] != 0


_mask_tc = pl.pallas_call(
    _mask_body,
    out_shape=jax.ShapeDtypeStruct((BATCH, SEQ), jnp.bool_),
)


@jax.jit
def kernel(x, word_emb, pos_emb):
    x_idx = x.reshape(NW, ROWS_PER_W, 2, HALF)
    out = _emb_lookup(x_idx, word_emb, pos_emb)
    mask = _mask_tc(x)
    return (out, mask)
